# baseline (speedup 1.0000x reference)
import functools

import jax
import jax.numpy as jnp
from jax import lax
from jax.experimental import pallas as pl
from jax.experimental.pallas import tpu as pltpu
from jax.experimental.pallas import tpu_sc as plsc

NC = 2
NS = 16
NW = NC * NS

CH = 32
NBUF = 10


def _ek(total_rows, d, nch):
    mesh = plsc.VectorSubcoreMesh(core_axis_name="c", subcore_axis_name="s")
    b_per_w = nch * CH
    assert nch % NBUF == 0 and nch // NBUF >= 2

    @functools.partial(
        pl.kernel,
        mesh=mesh,
        out_type=jax.ShapeDtypeStruct((total_rows, d), jnp.float32),
        scratch_types=(
            [pltpu.VMEM((nch, CH), jnp.int32)]
            + [pltpu.VMEM((CH, d), jnp.float32) for _ in range(NBUF)]
            + [pltpu.SemaphoreType.DMA for _ in range(2 * NBUF)]
        ),
    )
    def k(idx_hbm, table_hbm, out_hbm, idx_v, *rest):
        bufs = rest[:NBUF]
        gs = rest[NBUF:2 * NBUF]
        ss = rest[2 * NBUF:]
        wid = lax.axis_index("s") * NC + lax.axis_index("c")
        base = wid * b_per_w

        pltpu.sync_copy(idx_hbm.at[wid], idx_v)

        def gather(c, b):
            return pltpu.make_async_copy(table_hbm.at[idx_v.at[c]], bufs[b], gs[b])

        def scatter(c, b):
            return pltpu.make_async_copy(
                bufs[b], out_hbm.at[pl.ds(base + c * CH, CH)], ss[b])

        for b in range(NBUF):
            gather(b, b).start()

        def body(p, _):
            c = NBUF * p
            for b in range(NBUF):
                gather(c + b, b).wait()
                scatter(c + b, b).start()
            for b in range(NBUF):
                scatter(c + b, b).wait()
                gather(c + NBUF + b, b).start()
            return 0

        lax.fori_loop(0, nch // NBUF - 1, body, 0, unroll=False)

        c = nch - NBUF
        for b in range(NBUF):
            gather(c + b, b).wait()
            scatter(c + b, b).start()
        for b in range(NBUF):
            scatter(c + b, b).wait()

    return k


def kernel(indices, weight):
    b, s = indices.shape
    v, d = weight.shape
    total = b * s
    assert total % (NW * CH) == 0
    nch = total // (NW * CH)
    idx = indices.reshape(NW, nch, CH)
    out = _ek(total, d, nch)(idx, weight)
    return out.reshape(b, s, d)


# speedup vs baseline: 1.0565x; 1.0565x over previous
import functools

import jax
import jax.numpy as jnp
from jax import lax
from jax.experimental import pallas as pl
from jax.experimental.pallas import tpu as pltpu
from jax.experimental.pallas import tpu_sc as plsc

NC = 2
NS = 16
NW = NC * NS

CH = 64
NBUF = 10


def _ek(total_rows, d, nch):
    mesh = plsc.VectorSubcoreMesh(core_axis_name="c", subcore_axis_name="s")
    b_per_w = nch * CH
    assert nch % NBUF == 0 and nch // NBUF >= 2

    @functools.partial(
        pl.kernel,
        mesh=mesh,
        out_type=jax.ShapeDtypeStruct((total_rows, d), jnp.float32),
        scratch_types=(
            [pltpu.VMEM((nch, CH), jnp.int32)]
            + [pltpu.VMEM((CH, d), jnp.float32) for _ in range(NBUF)]
            + [pltpu.SemaphoreType.DMA for _ in range(2 * NBUF)]
        ),
    )
    def k(idx_hbm, table_hbm, out_hbm, idx_v, *rest):
        bufs = rest[:NBUF]
        gs = rest[NBUF:2 * NBUF]
        ss = rest[2 * NBUF:]
        wid = lax.axis_index("s") * NC + lax.axis_index("c")
        base = wid * b_per_w

        pltpu.sync_copy(idx_hbm.at[wid], idx_v)

        def gather(c, b):
            return pltpu.make_async_copy(table_hbm.at[idx_v.at[c]], bufs[b], gs[b])

        def scatter(c, b):
            return pltpu.make_async_copy(
                bufs[b], out_hbm.at[pl.ds(base + c * CH, CH)], ss[b])

        for b in range(NBUF):
            gather(b, b).start()

        def body(p, _):
            c = NBUF * p
            for b in range(NBUF):
                gather(c + b, b).wait()
                scatter(c + b, b).start()
            for b in range(NBUF):
                scatter(c + b, b).wait()
                gather(c + NBUF + b, b).start()
            return 0

        lax.fori_loop(0, nch // NBUF - 1, body, 0, unroll=False)

        c = nch - NBUF
        for b in range(NBUF):
            gather(c + b, b).wait()
            scatter(c + b, b).start()
        for b in range(NBUF):
            scatter(c + b, b).wait()

    return k


def kernel(indices, weight):
    b, s = indices.shape
    v, d = weight.shape
    total = b * s
    assert total % (NW * CH) == 0
    nch = total // (NW * CH)
    idx = indices.reshape(NW, nch, CH)
    out = _ek(total, d, nch)(idx, weight)
    return out.reshape(b, s, d)
